# baseline (speedup 1.0000x reference)
import functools

import jax
import jax.numpy as jnp
from jax import lax
from jax.experimental import pallas as pl
from jax.experimental.pallas import tpu as pltpu
from jax.experimental.pallas import tpu_sc as plsc

N = 10000
NPAD = 10240
E = 320000
NFEAT = 128
NHID = 64
NCLASS = 16

NUM_CORES = 2
NUM_SUBCORES = 16
NUM_TILES = NUM_CORES * NUM_SUBCORES

CHUNK = 128
NCHUNKS = E // CHUNK
BASE_CHUNKS = NCHUNKS // NUM_TILES
EXTRA_BASE = BASE_CHUNKS * NUM_TILES
N_EXTRA = NCHUNKS - EXTRA_BASE
ROWS_PER_SUBCORE = NPAD // NUM_SUBCORES

_BLK = 1024
_NBLK = 10

_f32 = jnp.float32


_sc_mesh = plsc.VectorSubcoreMesh(core_axis_name="c", subcore_axis_name="s")


_PHALF = NPAD // 2


@functools.partial(
    pl.kernel,
    mesh=_sc_mesh,
    out_type=[
        jax.ShapeDtypeStruct((NUM_CORES, NPAD), _f32),
        jax.ShapeDtypeStruct((NUM_CORES, NPAD), _f32),
    ],
    compiler_params=pltpu.CompilerParams(use_tc_tiling_on_sc=False),
    scratch_types=[
        pltpu.VMEM((BASE_CHUNKS + 1, 2, CHUNK), jnp.int32),
        pltpu.VMEM((BASE_CHUNKS + 1, CHUNK), jnp.int32),
        pltpu.VMEM((CHUNK,), _f32),
        pltpu.VMEM((ROWS_PER_SUBCORE,), _f32),
        pltpu.VMEM_SHARED((NPAD,), _f32),
        pltpu.VMEM_SHARED((NPAD,), _f32),
        pltpu.SemaphoreType.DMA,
        pltpu.SemaphoreType.DMA,
        pltpu.SemaphoreType.DMA,
        pltpu.SemaphoreType.DMA,
    ],
)
def _sc_hist(eil_hbm, out_hbm, outp_hbm, eil_v, ps_v, ones_v, zero_v,
             acc_sh, accp_sh, sem_a, sem_b, sem_c, sem_d):
    c = lax.axis_index("c")
    s = lax.axis_index("s")
    wid = s * NUM_CORES + c

    ones16 = jnp.ones((16,), _f32)
    zeros16 = jnp.zeros((16,), _f32)
    for i in range(CHUNK // 16):
        ones_v[pl.ds(i * 16, 16)] = ones16
    for i in range(ROWS_PER_SUBCORE // 16):
        zero_v[pl.ds(i * 16, 16)] = zeros16

    pltpu.sync_copy(zero_v, acc_sh.at[pl.ds(s * ROWS_PER_SUBCORE, ROWS_PER_SUBCORE)])
    pltpu.sync_copy(zero_v, accp_sh.at[pl.ds(s * ROWS_PER_SUBCORE, ROWS_PER_SUBCORE)])
    plsc.subcore_barrier()

    pltpu.sync_copy(eil_hbm.at[pl.ds(wid * BASE_CHUNKS, BASE_CHUNKS)],
                    eil_v.at[pl.ds(0, BASE_CHUNKS)])

    @pl.when(wid < N_EXTRA)
    def _():
        pltpu.sync_copy(eil_hbm.at[pl.ds(EXTRA_BASE + wid, 1)],
                        eil_v.at[pl.ds(BASE_CHUNKS, 1)])

    def compute_ps(j):
        for i in range(CHUNK // 16):
            v = eil_v[j, 1, pl.ds(i * 16, 16)]
            ps_v[j, pl.ds(i * 16, 16)] = (
                (v & 1) * _PHALF + lax.shift_right_logical(v, 1)
            )

    def issue(j, sa, sb):
        pltpu.async_copy(ones_v, acc_sh.at[eil_v.at[j, 1]], sa, add=True)
        pltpu.async_copy(ones_v, accp_sh.at[ps_v.at[j]], sb, add=True)

    def drain(j, sa, sb):
        pltpu.make_async_copy(ones_v, acc_sh.at[eil_v.at[j, 1]], sa).wait()
        pltpu.make_async_copy(ones_v, accp_sh.at[ps_v.at[j]], sb).wait()

    compute_ps(0)
    compute_ps(1)
    compute_ps(2)
    issue(0, sem_a, sem_b)

    def body(t, carry):
        issue(2 * t + 1, sem_c, sem_d)
        compute_ps(2 * t + 3)
        drain(2 * t, sem_a, sem_b)
        issue(2 * t + 2, sem_a, sem_b)
        compute_ps(2 * t + 4)
        drain(2 * t + 1, sem_c, sem_d)
        return carry

    lax.fori_loop(0, BASE_CHUNKS // 2 - 1, body, 0)
    issue(BASE_CHUNKS - 1, sem_c, sem_d)
    drain(BASE_CHUNKS - 2, sem_a, sem_b)
    drain(BASE_CHUNKS - 1, sem_c, sem_d)

    @pl.when(wid < N_EXTRA)
    def _():
        compute_ps(BASE_CHUNKS)
        pltpu.sync_copy(ones_v, acc_sh.at[eil_v.at[BASE_CHUNKS, 1]], add=True)
        pltpu.sync_copy(ones_v, accp_sh.at[ps_v.at[BASE_CHUNKS]], add=True)

    plsc.subcore_barrier()

    pltpu.sync_copy(
        acc_sh.at[pl.ds(s * ROWS_PER_SUBCORE, ROWS_PER_SUBCORE)],
        out_hbm.at[c, pl.ds(s * ROWS_PER_SUBCORE, ROWS_PER_SUBCORE)],
    )
    pltpu.sync_copy(
        accp_sh.at[pl.ds(s * ROWS_PER_SUBCORE, ROWS_PER_SUBCORE)],
        outp_hbm.at[c, pl.ds(s * ROWS_PER_SUBCORE, ROWS_PER_SUBCORE)],
    )


_PIPE = BASE_CHUNKS // 4 * 4


@functools.partial(
    pl.kernel,
    mesh=_sc_mesh,
    out_type=jax.ShapeDtypeStruct((NUM_CORES, NPAD, NHID), _f32),
    compiler_params=pltpu.CompilerParams(use_tc_tiling_on_sc=False),
    scratch_types=[
        pltpu.VMEM((BASE_CHUNKS + 1, 2, CHUNK), jnp.int32),
        pltpu.VMEM((4, CHUNK, NHID), _f32),
        pltpu.VMEM_SHARED((NPAD, NHID), _f32),
        pltpu.SemaphoreType.DMA,
        pltpu.SemaphoreType.DMA,
        pltpu.SemaphoreType.DMA,
        pltpu.SemaphoreType.DMA,
        pltpu.SemaphoreType.DMA,
        pltpu.SemaphoreType.DMA,
        pltpu.SemaphoreType.DMA,
        pltpu.SemaphoreType.DMA,
    ],
)
def _sc_scatter(
    g_hbm, eil_hbm, out_hbm, eil_v, rows_v,
    acc_sh, g0, g1, g2, g3, s0, s1, s2, s3
):
    c = lax.axis_index("c")
    s = lax.axis_index("s")
    wid = s * NUM_CORES + c
    gsem = (g0, g1, g2, g3)
    ssem = (s0, s1, s2, s3)

    zeros16 = jnp.zeros((16,), _f32)
    for r in range(CHUNK):
        for k in range(NHID // 16):
            rows_v[0, r, pl.ds(k * 16, 16)] = zeros16
    for k in range(ROWS_PER_SUBCORE // CHUNK):
        pltpu.sync_copy(
            rows_v.at[0], acc_sh.at[pl.ds(s * ROWS_PER_SUBCORE + k * CHUNK, CHUNK)]
        )
    plsc.subcore_barrier()

    pltpu.sync_copy(eil_hbm.at[pl.ds(wid * BASE_CHUNKS, BASE_CHUNKS)],
                    eil_v.at[pl.ds(0, BASE_CHUNKS)])

    @pl.when(wid < N_EXTRA)
    def _():
        pltpu.sync_copy(eil_hbm.at[pl.ds(EXTRA_BASE + wid, 1)],
                        eil_v.at[pl.ds(BASE_CHUNKS, 1)])

    for k in range(4):
        pltpu.async_copy(g_hbm.at[eil_v.at[k, 0]], rows_v.at[k], gsem[k])

    def body(t, carry):
        for k in range(4):
            j = 4 * t + k
            pltpu.make_async_copy(g_hbm.at[eil_v.at[j, 0]], rows_v.at[k], gsem[k]).wait()
            pltpu.async_copy(rows_v.at[k], acc_sh.at[eil_v.at[j, 1]], ssem[k], add=True)
        for k in range(4):
            j = 4 * t + k
            jn = jnp.minimum(j + 4, _PIPE + 1)
            pltpu.make_async_copy(rows_v.at[k], acc_sh.at[eil_v.at[j, 1]], ssem[k]).wait()
            pltpu.async_copy(g_hbm.at[eil_v.at[jn, 0]], rows_v.at[k], gsem[k])
        return carry

    lax.fori_loop(0, _PIPE // 4, body, 0)
    for k in range(4):
        pltpu.make_async_copy(
            g_hbm.at[eil_v.at[_PIPE + (k if k < 2 else 1), 0]], rows_v.at[k], gsem[k]
        ).wait()
    for k in range(2):
        pltpu.sync_copy(rows_v.at[k], acc_sh.at[eil_v.at[_PIPE + k, 1]], add=True)

    @pl.when(wid < N_EXTRA)
    def _():
        pltpu.async_copy(
            g_hbm.at[eil_v.at[BASE_CHUNKS, 0]], rows_v.at[2], g2
        ).wait()
        pltpu.sync_copy(rows_v.at[2], acc_sh.at[eil_v.at[BASE_CHUNKS, 1]], add=True)

    plsc.subcore_barrier()

    pltpu.sync_copy(
        acc_sh.at[pl.ds(s * ROWS_PER_SUBCORE, ROWS_PER_SUBCORE)],
        out_hbm.at[c, pl.ds(s * ROWS_PER_SUBCORE, ROWS_PER_SUBCORE)],
    )


def _tc_h_body(x_ref, w_ref, h_ref):
    h_ref[...] = jnp.dot(x_ref[...], w_ref[...], preferred_element_type=_f32)


def _tc_h(x, W1):
    return pl.pallas_call(
        _tc_h_body,
        grid=(_NBLK,),
        in_specs=[
            pl.BlockSpec((_BLK, NFEAT), lambda i: (i, 0)),
            pl.BlockSpec((NFEAT, NHID), lambda i: (0, 0)),
        ],
        out_specs=pl.BlockSpec((_BLK, NHID), lambda i: (i, 0)),
        out_shape=jax.ShapeDtypeStruct((N, NHID), _f32),
    )(x, W1)


def _tc_scale_body(h_ref, d0_ref, d1_ref, g_ref):
    deg = d0_ref[...] + d1_ref[...] + 1.0
    dinv = lax.rsqrt(deg)
    g_ref[...] = h_ref[...] * dinv[:, None]


def _tc_scale(h, degp_flat):
    return pl.pallas_call(
        _tc_scale_body,
        grid=(_NBLK,),
        in_specs=[
            pl.BlockSpec((_BLK, NHID), lambda i: (i, 0)),
            pl.BlockSpec((_BLK,), lambda i: (i,)),
            pl.BlockSpec((_BLK,), lambda i: (i + _NBLK,)),
        ],
        out_specs=pl.BlockSpec((_BLK, NHID), lambda i: (i, 0)),
        out_shape=jax.ShapeDtypeStruct((N, NHID), _f32),
    )(h, degp_flat, degp_flat)


_BLKP = _BLK // 2


def _tc_b_body(s_ref, g_ref, d0e, d0o, d1e, d1o, b1_ref, wc_ref, bc_ref, o_ref):
    dinv_e = lax.rsqrt(d0e[...] + d1e[...] + 1.0)
    dinv_o = lax.rsqrt(d0o[...] + d1o[...] + 1.0)
    dinv128 = jnp.concatenate(
        [
            jnp.broadcast_to(dinv_e[:, None], (_BLKP, NHID)),
            jnp.broadcast_to(dinv_o[:, None], (_BLKP, NHID)),
        ],
        axis=1,
    )
    tot = s_ref[0] + s_ref[1] + g_ref[...]
    pre = tot * dinv128 + b1_ref[...]
    h2 = jnp.maximum(pre, 0.0)
    logits = jnp.dot(h2, wc_ref[...], preferred_element_type=_f32) + bc_ref[...]
    l0 = logits[:, :NCLASS]
    l1 = logits[:, NCLASS:]
    z0 = l0 - jnp.max(l0, axis=1, keepdims=True)
    z1 = l1 - jnp.max(l1, axis=1, keepdims=True)
    z0 = z0 - jnp.log(jnp.sum(jnp.exp(z0), axis=1, keepdims=True))
    z1 = z1 - jnp.log(jnp.sum(jnp.exp(z1), axis=1, keepdims=True))
    o_ref[...] = jnp.concatenate([z0, z1], axis=1)


def _tc_b(S2, g2, degps_flat, b1_2, Wc2, bc2):
    nq = _PHALF // _BLKP
    return pl.pallas_call(
        _tc_b_body,
        grid=(_NBLK,),
        in_specs=[
            pl.BlockSpec((2, _BLKP, 2 * NHID), lambda i: (0, i, 0)),
            pl.BlockSpec((_BLKP, 2 * NHID), lambda i: (i, 0)),
            pl.BlockSpec((_BLKP,), lambda i: (i,)),
            pl.BlockSpec((_BLKP,), lambda i: (i + nq,)),
            pl.BlockSpec((_BLKP,), lambda i: (i + 2 * nq,)),
            pl.BlockSpec((_BLKP,), lambda i: (i + 3 * nq,)),
            pl.BlockSpec((1, 2 * NHID), lambda i: (0, 0)),
            pl.BlockSpec((2 * NHID, 2 * NCLASS), lambda i: (0, 0)),
            pl.BlockSpec((1, 2 * NCLASS), lambda i: (0, 0)),
        ],
        out_specs=pl.BlockSpec((_BLKP, 2 * NCLASS), lambda i: (i, 0)),
        out_shape=jax.ShapeDtypeStruct((N // 2, 2 * NCLASS), _f32),
    )(S2, g2, degps_flat, degps_flat, degps_flat, degps_flat, b1_2, Wc2, bc2)


def kernel(x, edge_index, W1, b1, Wc, bc):
    eil = edge_index.reshape(2, NCHUNKS, CHUNK).transpose(1, 0, 2)

    degp, degps = _sc_hist(eil)
    degp_flat = degp.reshape(-1)
    h = _tc_h(x, W1)
    g = _tc_scale(h, degp_flat)
    g_lin = lax.optimization_barrier(g.reshape(-1))
    S = _sc_scatter(g_lin.reshape(N, NHID), eil)

    S2 = S.reshape(NUM_CORES, NPAD // 2, 2 * NHID)
    g2 = g_lin.reshape(N // 2, 2 * NHID)
    b1_2 = jnp.concatenate([b1, b1]).reshape(1, 2 * NHID)
    bc2 = jnp.concatenate([bc, bc]).reshape(1, 2 * NCLASS)
    z = jnp.zeros((NHID, NCLASS), _f32)
    Wc2 = jnp.block([[Wc, z], [z, Wc]])
    out2 = _tc_b(S2, g2, degps.reshape(-1), b1_2, Wc2, bc2)
    return out2.reshape(N, NCLASS)


# speedup vs baseline: 1.0181x; 1.0181x over previous
import functools

import jax
import jax.numpy as jnp
from jax import lax
from jax.experimental import pallas as pl
from jax.experimental.pallas import tpu as pltpu
from jax.experimental.pallas import tpu_sc as plsc

N = 10000
NPAD = 10240
E = 320000
NFEAT = 128
NHID = 64
NCLASS = 16

NUM_CORES = 2
NUM_SUBCORES = 16
NUM_TILES = NUM_CORES * NUM_SUBCORES

CHUNK = 128
NCHUNKS = E // CHUNK
BASE_CHUNKS = NCHUNKS // NUM_TILES
EXTRA_BASE = BASE_CHUNKS * NUM_TILES
N_EXTRA = NCHUNKS - EXTRA_BASE
ROWS_PER_SUBCORE = NPAD // NUM_SUBCORES

_BLK = 1024
_NBLK = 10

_f32 = jnp.float32


_sc_mesh = plsc.VectorSubcoreMesh(core_axis_name="c", subcore_axis_name="s")


_PHALF = NPAD // 2


@functools.partial(
    pl.kernel,
    mesh=_sc_mesh,
    out_type=[
        jax.ShapeDtypeStruct((NUM_CORES, NPAD), _f32),
        jax.ShapeDtypeStruct((NUM_CORES, NPAD), _f32),
    ],
    compiler_params=pltpu.CompilerParams(use_tc_tiling_on_sc=False),
    scratch_types=[
        pltpu.VMEM((BASE_CHUNKS + 1, 2, CHUNK), jnp.int32),
        pltpu.VMEM((BASE_CHUNKS + 1, CHUNK), jnp.int32),
        pltpu.VMEM((CHUNK,), _f32),
        pltpu.VMEM((ROWS_PER_SUBCORE,), _f32),
        pltpu.VMEM_SHARED((NPAD,), _f32),
        pltpu.VMEM_SHARED((NPAD,), _f32),
        pltpu.SemaphoreType.DMA,
        pltpu.SemaphoreType.DMA,
        pltpu.SemaphoreType.DMA,
        pltpu.SemaphoreType.DMA,
    ],
)
def _sc_hist(eil_hbm, out_hbm, outp_hbm, eil_v, ps_v, ones_v, zero_v,
             acc_sh, accp_sh, sem_a, sem_b, sem_c, sem_d):
    c = lax.axis_index("c")
    s = lax.axis_index("s")
    wid = s * NUM_CORES + c

    ones16 = jnp.ones((16,), _f32)
    zeros16 = jnp.zeros((16,), _f32)
    for i in range(CHUNK // 16):
        ones_v[pl.ds(i * 16, 16)] = ones16
    for i in range(ROWS_PER_SUBCORE // 16):
        zero_v[pl.ds(i * 16, 16)] = zeros16

    pltpu.sync_copy(zero_v, acc_sh.at[pl.ds(s * ROWS_PER_SUBCORE, ROWS_PER_SUBCORE)])
    pltpu.sync_copy(zero_v, accp_sh.at[pl.ds(s * ROWS_PER_SUBCORE, ROWS_PER_SUBCORE)])
    plsc.subcore_barrier()

    pltpu.sync_copy(eil_hbm.at[pl.ds(wid * BASE_CHUNKS, BASE_CHUNKS)],
                    eil_v.at[pl.ds(0, BASE_CHUNKS)])

    @pl.when(wid < N_EXTRA)
    def _():
        pltpu.sync_copy(eil_hbm.at[pl.ds(EXTRA_BASE + wid, 1)],
                        eil_v.at[pl.ds(BASE_CHUNKS, 1)])

    def compute_ps(j):
        for i in range(CHUNK // 16):
            v = eil_v[j, 1, pl.ds(i * 16, 16)]
            ps_v[j, pl.ds(i * 16, 16)] = (
                (v & 1) * _PHALF + lax.shift_right_logical(v, 1)
            )

    def issue(j, sa, sb):
        pltpu.async_copy(ones_v, acc_sh.at[eil_v.at[j, 1]], sa, add=True)
        pltpu.async_copy(ones_v, accp_sh.at[ps_v.at[j]], sb, add=True)

    def drain(j, sa, sb):
        pltpu.make_async_copy(ones_v, acc_sh.at[eil_v.at[j, 1]], sa).wait()
        pltpu.make_async_copy(ones_v, accp_sh.at[ps_v.at[j]], sb).wait()

    compute_ps(0)
    compute_ps(1)
    compute_ps(2)
    issue(0, sem_a, sem_b)

    def body(t, carry):
        issue(2 * t + 1, sem_c, sem_d)
        compute_ps(2 * t + 3)
        drain(2 * t, sem_a, sem_b)
        issue(2 * t + 2, sem_a, sem_b)
        compute_ps(2 * t + 4)
        drain(2 * t + 1, sem_c, sem_d)
        return carry

    lax.fori_loop(0, BASE_CHUNKS // 2 - 1, body, 0)
    issue(BASE_CHUNKS - 1, sem_c, sem_d)
    drain(BASE_CHUNKS - 2, sem_a, sem_b)
    drain(BASE_CHUNKS - 1, sem_c, sem_d)

    @pl.when(wid < N_EXTRA)
    def _():
        compute_ps(BASE_CHUNKS)
        pltpu.sync_copy(ones_v, acc_sh.at[eil_v.at[BASE_CHUNKS, 1]], add=True)
        pltpu.sync_copy(ones_v, accp_sh.at[ps_v.at[BASE_CHUNKS]], add=True)

    plsc.subcore_barrier()

    pltpu.sync_copy(
        acc_sh.at[pl.ds(s * ROWS_PER_SUBCORE, ROWS_PER_SUBCORE)],
        out_hbm.at[c, pl.ds(s * ROWS_PER_SUBCORE, ROWS_PER_SUBCORE)],
    )
    pltpu.sync_copy(
        accp_sh.at[pl.ds(s * ROWS_PER_SUBCORE, ROWS_PER_SUBCORE)],
        outp_hbm.at[c, pl.ds(s * ROWS_PER_SUBCORE, ROWS_PER_SUBCORE)],
    )


_RING = 8
_PIPE = BASE_CHUNKS // _RING * _RING
_TAIL = BASE_CHUNKS - _PIPE


@functools.partial(
    pl.kernel,
    mesh=_sc_mesh,
    out_type=jax.ShapeDtypeStruct((NUM_CORES, NPAD, NHID), _f32),
    compiler_params=pltpu.CompilerParams(use_tc_tiling_on_sc=False),
    scratch_types=[
        pltpu.VMEM((BASE_CHUNKS + 1, 2, CHUNK), jnp.int32),
        pltpu.VMEM((_RING, CHUNK, NHID), _f32),
        pltpu.VMEM_SHARED((NPAD, NHID), _f32),
    ]
    + [pltpu.SemaphoreType.DMA] * (2 * _RING),
)
def _sc_scatter(
    g_hbm, eil_hbm, out_hbm, eil_v, rows_v, acc_sh, *sems
):
    c = lax.axis_index("c")
    s = lax.axis_index("s")
    wid = s * NUM_CORES + c
    gsem = sems[:_RING]
    ssem = sems[_RING:]

    zeros16 = jnp.zeros((16,), _f32)
    for r in range(CHUNK):
        for k in range(NHID // 16):
            rows_v[0, r, pl.ds(k * 16, 16)] = zeros16
    for k in range(ROWS_PER_SUBCORE // CHUNK):
        pltpu.sync_copy(
            rows_v.at[0], acc_sh.at[pl.ds(s * ROWS_PER_SUBCORE + k * CHUNK, CHUNK)]
        )
    plsc.subcore_barrier()

    pltpu.sync_copy(eil_hbm.at[pl.ds(wid * BASE_CHUNKS, BASE_CHUNKS)],
                    eil_v.at[pl.ds(0, BASE_CHUNKS)])

    @pl.when(wid < N_EXTRA)
    def _():
        pltpu.sync_copy(eil_hbm.at[pl.ds(EXTRA_BASE + wid, 1)],
                        eil_v.at[pl.ds(BASE_CHUNKS, 1)])

    for k in range(_RING):
        pltpu.async_copy(g_hbm.at[eil_v.at[k, 0]], rows_v.at[k], gsem[k])

    def body(t, carry):
        for k in range(_RING):
            j = _RING * t + k
            pltpu.make_async_copy(g_hbm.at[eil_v.at[j, 0]], rows_v.at[k], gsem[k]).wait()
            pltpu.async_copy(rows_v.at[k], acc_sh.at[eil_v.at[j, 1]], ssem[k], add=True)
        for k in range(_RING):
            j = _RING * t + k
            jn = jnp.minimum(j + _RING, BASE_CHUNKS - 1)
            pltpu.make_async_copy(rows_v.at[k], acc_sh.at[eil_v.at[j, 1]], ssem[k]).wait()
            pltpu.async_copy(g_hbm.at[eil_v.at[jn, 0]], rows_v.at[k], gsem[k])
        return carry

    lax.fori_loop(0, _PIPE // _RING, body, 0)
    for k in range(_RING):
        pltpu.make_async_copy(
            g_hbm.at[eil_v.at[_PIPE + min(k, _TAIL - 1), 0]], rows_v.at[k], gsem[k]
        ).wait()
    for k in range(_TAIL):
        pltpu.sync_copy(rows_v.at[k], acc_sh.at[eil_v.at[_PIPE + k, 1]], add=True)

    @pl.when(wid < N_EXTRA)
    def _():
        pltpu.async_copy(
            g_hbm.at[eil_v.at[BASE_CHUNKS, 0]], rows_v.at[_TAIL], gsem[_TAIL]
        ).wait()
        pltpu.sync_copy(rows_v.at[_TAIL], acc_sh.at[eil_v.at[BASE_CHUNKS, 1]], add=True)

    plsc.subcore_barrier()

    pltpu.sync_copy(
        acc_sh.at[pl.ds(s * ROWS_PER_SUBCORE, ROWS_PER_SUBCORE)],
        out_hbm.at[c, pl.ds(s * ROWS_PER_SUBCORE, ROWS_PER_SUBCORE)],
    )


def _tc_a_body(x_ref, w_ref, d0_ref, d1_ref, g_ref):
    deg = d0_ref[...] + d1_ref[...] + 1.0
    dinv = lax.rsqrt(deg)
    h = jnp.dot(x_ref[...], w_ref[...], preferred_element_type=_f32)
    g_ref[...] = h * dinv[:, None]


def _tc_a(x, W1, degp_flat):
    return pl.pallas_call(
        _tc_a_body,
        grid=(_NBLK,),
        in_specs=[
            pl.BlockSpec((_BLK, NFEAT), lambda i: (i, 0)),
            pl.BlockSpec((NFEAT, NHID), lambda i: (0, 0)),
            pl.BlockSpec((_BLK,), lambda i: (i,)),
            pl.BlockSpec((_BLK,), lambda i: (i + _NBLK,)),
        ],
        out_specs=pl.BlockSpec((_BLK, NHID), lambda i: (i, 0)),
        out_shape=jax.ShapeDtypeStruct((N, NHID), _f32),
    )(x, W1, degp_flat, degp_flat)


_BLKP = _BLK // 2


def _tc_b_body(s_ref, g_ref, d0e, d0o, d1e, d1o, b1_ref, wc_ref, bc_ref, o_ref):
    dinv_e = lax.rsqrt(d0e[...] + d1e[...] + 1.0)
    dinv_o = lax.rsqrt(d0o[...] + d1o[...] + 1.0)
    dinv128 = jnp.concatenate(
        [
            jnp.broadcast_to(dinv_e[:, None], (_BLKP, NHID)),
            jnp.broadcast_to(dinv_o[:, None], (_BLKP, NHID)),
        ],
        axis=1,
    )
    tot = s_ref[0] + s_ref[1] + g_ref[...]
    pre = tot * dinv128 + b1_ref[...]
    h2 = jnp.maximum(pre, 0.0)
    logits = jnp.dot(h2, wc_ref[...], preferred_element_type=_f32) + bc_ref[...]
    l0 = logits[:, :NCLASS]
    l1 = logits[:, NCLASS:]
    z0 = l0 - jnp.max(l0, axis=1, keepdims=True)
    z1 = l1 - jnp.max(l1, axis=1, keepdims=True)
    z0 = z0 - jnp.log(jnp.sum(jnp.exp(z0), axis=1, keepdims=True))
    z1 = z1 - jnp.log(jnp.sum(jnp.exp(z1), axis=1, keepdims=True))
    o_ref[...] = jnp.concatenate([z0, z1], axis=1)


def _tc_b(S2, g2, degps_flat, b1_2, Wc2, bc2):
    nq = _PHALF // _BLKP
    return pl.pallas_call(
        _tc_b_body,
        grid=(_NBLK,),
        in_specs=[
            pl.BlockSpec((2, _BLKP, 2 * NHID), lambda i: (0, i, 0)),
            pl.BlockSpec((_BLKP, 2 * NHID), lambda i: (i, 0)),
            pl.BlockSpec((_BLKP,), lambda i: (i,)),
            pl.BlockSpec((_BLKP,), lambda i: (i + nq,)),
            pl.BlockSpec((_BLKP,), lambda i: (i + 2 * nq,)),
            pl.BlockSpec((_BLKP,), lambda i: (i + 3 * nq,)),
            pl.BlockSpec((1, 2 * NHID), lambda i: (0, 0)),
            pl.BlockSpec((2 * NHID, 2 * NCLASS), lambda i: (0, 0)),
            pl.BlockSpec((1, 2 * NCLASS), lambda i: (0, 0)),
        ],
        out_specs=pl.BlockSpec((_BLKP, 2 * NCLASS), lambda i: (i, 0)),
        out_shape=jax.ShapeDtypeStruct((N // 2, 2 * NCLASS), _f32),
    )(S2, g2, degps_flat, degps_flat, degps_flat, degps_flat, b1_2, Wc2, bc2)


def kernel(x, edge_index, W1, b1, Wc, bc):
    eil = edge_index.reshape(2, NCHUNKS, CHUNK).transpose(1, 0, 2)

    degp, degps = _sc_hist(eil)
    degp_flat = degp.reshape(-1)
    g = _tc_a(x, W1, degp_flat)
    g_lin = lax.optimization_barrier(g.reshape(-1))
    S = _sc_scatter(g_lin.reshape(N, NHID), eil)

    S2 = S.reshape(NUM_CORES, NPAD // 2, 2 * NHID)
    g2 = g_lin.reshape(N // 2, 2 * NHID)
    b1_2 = jnp.concatenate([b1, b1]).reshape(1, 2 * NHID)
    bc2 = jnp.concatenate([bc, bc]).reshape(1, 2 * NCLASS)
    z = jnp.zeros((NHID, NCLASS), _f32)
    Wc2 = jnp.block([[Wc, z], [z, Wc]])
    out2 = _tc_b(S2, g2, degps.reshape(-1), b1_2, Wc2, bc2)
    return out2.reshape(N, NCLASS)
